# baseline (speedup 1.0000x reference)
import numpy as np
import jax
import jax.numpy as jnp
from jax.experimental import pallas as pl
from jax.experimental.pallas import tpu as pltpu

BT = 16
MT = 256


def _conv_kernel(x_ref, w1_ref, w2_ref, w3_ref, b1_ref, b2_ref, b3_ref,
                 o_ref, xt1, acc1, xt2, acc2, xt3, acc3):
    f32 = jnp.float32

    for c in range(3):
        xt1[:, :, 64 * c:64 * (c + 1)] = x_ref[:, c, :, :]

    def pool_bias_relu(acc, rows, bias):
        a = acc[...].reshape(rows, 2, 896)
        rp = jnp.maximum(a[:, 0, :], a[:, 1, :])
        sh = jnp.concatenate([rp[:, 1:], rp[:, :1]], axis=-1)
        wm = jnp.maximum(rp, sh)
        return jnp.maximum(wm + bias, 0.0).astype(jnp.bfloat16)

    for dy in range(3):
        xs = xt1[:, dy:dy + 56, :].reshape(BT * 56, 192)
        d = jnp.dot(xs, w1_ref[dy], preferred_element_type=f32)
        if dy == 0:
            acc1[...] = d
        else:
            acc1[...] = acc1[...] + d
    act1 = pool_bias_relu(acc1, BT * 28, b1_ref[...])
    xt2[:, 0:1, :] = jnp.zeros((BT, 1, 896), jnp.bfloat16)
    xt2[:, 29:30, :] = jnp.zeros((BT, 1, 896), jnp.bfloat16)
    xt2[:, 1:29, :] = act1.reshape(BT, 28, 896)

    for dy in range(3):
        xs = xt2[:, dy:dy + 28, :].reshape(BT * 28, 896)
        d = jnp.dot(xs, w2_ref[dy], preferred_element_type=f32)
        if dy == 0:
            acc2[...] = d
        else:
            acc2[...] = acc2[...] + d
    act2 = pool_bias_relu(acc2, BT * 14, b2_ref[...])
    xt3[:, 0:1, :] = jnp.zeros((BT, 1, 896), jnp.bfloat16)
    xt3[:, 15:16, :] = jnp.zeros((BT, 1, 896), jnp.bfloat16)
    xt3[:, 1:15, :] = act2.reshape(BT, 14, 896)

    for dy in range(3):
        xs = xt3[:, dy:dy + 14, :].reshape(BT * 14, 896)
        d = jnp.dot(xs, w3_ref[dy], preferred_element_type=f32)
        if dy == 0:
            acc3[...] = d
        else:
            acc3[...] = acc3[...] + d
    act3 = pool_bias_relu(acc3, BT * 7, b3_ref[...])
    o_ref[...] = act3.reshape(BT, 7, 896)


def _conv_stack(xpad, w1t, w2t, w3t, b1t, b2t, b3t):
    B = xpad.shape[0]
    return pl.pallas_call(
        _conv_kernel,
        out_shape=jax.ShapeDtypeStruct((B, 7, 896), jnp.bfloat16),
        grid=(B // BT,),
        in_specs=[
            pl.BlockSpec((BT, 3, 58, 64), lambda b: (b, 0, 0, 0)),
            pl.BlockSpec((3, 192, 896), lambda b: (0, 0, 0)),
            pl.BlockSpec((3, 896, 896), lambda b: (0, 0, 0)),
            pl.BlockSpec((3, 896, 896), lambda b: (0, 0, 0)),
            pl.BlockSpec((1, 896), lambda b: (0, 0)),
            pl.BlockSpec((1, 896), lambda b: (0, 0)),
            pl.BlockSpec((1, 896), lambda b: (0, 0)),
        ],
        out_specs=pl.BlockSpec((BT, 7, 896), lambda b: (b, 0, 0)),
        scratch_shapes=[
            pltpu.VMEM((BT, 58, 192), jnp.bfloat16),
            pltpu.VMEM((BT * 56, 896), jnp.float32),
            pltpu.VMEM((BT, 30, 896), jnp.bfloat16),
            pltpu.VMEM((BT * 28, 896), jnp.float32),
            pltpu.VMEM((BT, 16, 896), jnp.bfloat16),
            pltpu.VMEM((BT * 14, 896), jnp.float32),
        ],
        compiler_params=pltpu.CompilerParams(
            dimension_semantics=("parallel",),
            vmem_limit_bytes=100 * 1024 * 1024),
    )(xpad, w1t, w2t, w3t, b1t, b2t, b3t)


def _mlp_kernel(x_ref, w1_ref, b1_ref, w2_ref, b2_ref, o_ref, acc):
    for i in range(7):
        d = jnp.dot(x_ref[:, i, :], w1_ref[i],
                    preferred_element_type=jnp.float32)
        if i == 0:
            acc[...] = d
        else:
            acc[...] = acc[...] + d
    h = jnp.maximum(acc[...] + b1_ref[...], 0.0).astype(jnp.bfloat16)
    o = jnp.dot(h, w2_ref[...], preferred_element_type=jnp.float32)
    o_ref[...] = o + b2_ref[...]


def _mlp(feat, w1m, b1, w2, b2):
    B = feat.shape[0]
    mt = min(MT, B)
    return pl.pallas_call(
        _mlp_kernel,
        out_shape=jax.ShapeDtypeStruct((B, 128), jnp.float32),
        grid=(B // mt,),
        in_specs=[
            pl.BlockSpec((mt, 7, 896), lambda b: (b, 0, 0)),
            pl.BlockSpec((7, 896, 128), lambda b: (0, 0, 0)),
            pl.BlockSpec((1, 128), lambda b: (0, 0)),
            pl.BlockSpec((128, 128), lambda b: (0, 0)),
            pl.BlockSpec((1, 128), lambda b: (0, 0)),
        ],
        out_specs=pl.BlockSpec((mt, 128), lambda b: (b, 0)),
        scratch_shapes=[pltpu.VMEM((mt, 128), jnp.float32)],
        compiler_params=pltpu.CompilerParams(
            dimension_semantics=("parallel",),
            vmem_limit_bytes=100 * 1024 * 1024),
    )(feat, w1m, b1, w2, b2)


def _toeplitz(wp, cin, cout, win, wout, kstride, cstride, interleave):
    eye = np.stack([np.eye(win, wout, k=1 - dx, dtype=np.float32)
                    for dx in range(3)])
    eye = jnp.asarray(eye)
    slabs = []
    for dy in range(3):
        w = wp[3 * dy:3 * dy + 3, :cin, :cout]
        t = jnp.einsum('duw,dcn->cunw', eye, w)
        if interleave:
            t = jnp.stack([t, jnp.zeros_like(t)], axis=2)
            t = t.reshape(cin, 2 * win, cout, wout)
        if cstride > t.shape[1]:
            t = jnp.pad(t, ((0, 0), (0, cstride - t.shape[1]), (0, 0), (0, 0)))
        slabs.append(t.reshape(cin * cstride, cout * wout))
    return jnp.stack(slabs).astype(jnp.bfloat16)


@jax.jit
def _forward(x_nchw, w1p, w2p, w3p, bstack, w1_fc, b1, w2_fc, b2):
    B = x_nchw.shape[0]
    w1t = _toeplitz(w1p, 3, 16, 56, 56, 1, 64, False)
    w2t = _toeplitz(w2p, 16, 32, 28, 28, 2, 56, True)
    w3t = _toeplitz(w3p, 32, 64, 14, 14, 2, 28, True)
    b1t = jnp.repeat(bstack[0, 0, 0, :16], 56).reshape(1, 896)
    b2t = jnp.repeat(bstack[1, 0, 0, :32], 28).reshape(1, 896)
    b3t = jnp.repeat(bstack[2, 0, 0, :64], 14).reshape(1, 896)
    f1 = w1_fc.reshape(7, 7, 128, 128)[:, :, :64, :]
    f1 = jnp.transpose(f1, (0, 2, 1, 3))
    f1 = jnp.stack([f1, jnp.zeros_like(f1)], axis=3)
    w1m = f1.reshape(7, 896, 128).astype(jnp.bfloat16)

    xpad = jnp.pad(x_nchw.astype(jnp.bfloat16),
                   ((0, 0), (0, 0), (1, 1), (0, 8)))
    feat = _conv_stack(xpad, w1t, w2t, w3t, b1t, b2t, b3t)
    out = _mlp(feat, w1m, b1, w2_fc.astype(jnp.bfloat16), b2)
    return out[:, :5]


def kernel(x_nchw, w1p, w2p, w3p, bstack, w1_fc, b1, w2_fc, b2):
    return _forward(x_nchw, w1p, w2p, w3p, bstack, w1_fc, b1, w2_fc, b2)


# speedup vs baseline: 1.4440x; 1.4440x over previous
import numpy as np
import jax
import jax.numpy as jnp
from jax.experimental import pallas as pl
from jax.experimental.pallas import tpu as pltpu

BT = 16
MT = 256


def _conv_kernel(x_ref, w1_ref, w2_ref, w3_ref, b1_ref, b2_ref, b3_ref,
                 o_ref, xt1, acc1, xt2, acc2, xt3, acc3):
    f32 = jnp.float32
    M = BT * 7

    for c in range(3):
        xt1[:, :, :, 64 * c:64 * (c + 1)] = x_ref[:, c, :, :, :]

    def conv(xt, nq, w_ref, acc, kdim):
        for dy in range(3):
            parts = []
            for q in range(nq):
                o = q + dy
                parts.append(xt[:, o % nq, o // nq:o // nq + 7, :])
            lhs = jnp.concatenate(parts, axis=0).reshape(nq * M, kdim)
            d = jnp.dot(lhs, w_ref[dy], preferred_element_type=f32)
            if dy == 0:
                acc[...] = d
            else:
                acc[...] = acc[...] + d
        return acc[...]

    def pool_bias_relu(a, c, bias):
        rp = jnp.maximum(a[2 * c * M:(2 * c + 1) * M, :],
                         a[(2 * c + 1) * M:(2 * c + 2) * M, :])
        sh = jnp.concatenate([rp[:, 1:], rp[:, :1]], axis=-1)
        wm = jnp.maximum(rp, sh)
        act = jnp.maximum(wm + bias, 0.0).astype(jnp.bfloat16)
        return act.reshape(BT, 7, 896)

    a1 = conv(xt1, 8, w1_ref, acc1, 192)
    xt2[:, 0, 0:1, :] = jnp.zeros((BT, 1, 896), jnp.bfloat16)
    xt2[:, 1, 7:8, :] = jnp.zeros((BT, 1, 896), jnp.bfloat16)
    for c in range(4):
        xt2[:, (c + 1) % 4, (c + 1) // 4:(c + 1) // 4 + 7, :] = \
            pool_bias_relu(a1, c, b1_ref[...])

    a2 = conv(xt2, 4, w2_ref, acc2, 896)
    xt3[:, 0, 0:1, :] = jnp.zeros((BT, 1, 896), jnp.bfloat16)
    xt3[:, 1, 7:8, :] = jnp.zeros((BT, 1, 896), jnp.bfloat16)
    for c in range(2):
        xt3[:, (c + 1) % 2, (c + 1) // 2:(c + 1) // 2 + 7, :] = \
            pool_bias_relu(a2, c, b2_ref[...])

    a3 = conv(xt3, 2, w3_ref, acc3, 896)
    o_ref[...] = pool_bias_relu(a3, 0, b3_ref[...])


def _conv_stack(xcl, w1t, w2t, w3t, b1t, b2t, b3t):
    B = xcl.shape[0]
    return pl.pallas_call(
        _conv_kernel,
        out_shape=jax.ShapeDtypeStruct((B, 7, 896), jnp.bfloat16),
        grid=(B // BT,),
        in_specs=[
            pl.BlockSpec((BT, 3, 8, 8, 64), lambda b: (b, 0, 0, 0, 0)),
            pl.BlockSpec((3, 192, 896), lambda b: (0, 0, 0)),
            pl.BlockSpec((3, 896, 896), lambda b: (0, 0, 0)),
            pl.BlockSpec((3, 896, 896), lambda b: (0, 0, 0)),
            pl.BlockSpec((1, 896), lambda b: (0, 0)),
            pl.BlockSpec((1, 896), lambda b: (0, 0)),
            pl.BlockSpec((1, 896), lambda b: (0, 0)),
        ],
        out_specs=pl.BlockSpec((BT, 7, 896), lambda b: (b, 0, 0)),
        scratch_shapes=[
            pltpu.VMEM((BT, 8, 8, 192), jnp.bfloat16),
            pltpu.VMEM((8 * BT * 7, 896), jnp.float32),
            pltpu.VMEM((BT, 4, 8, 896), jnp.bfloat16),
            pltpu.VMEM((4 * BT * 7, 896), jnp.float32),
            pltpu.VMEM((BT, 2, 8, 896), jnp.bfloat16),
            pltpu.VMEM((2 * BT * 7, 896), jnp.float32),
        ],
        compiler_params=pltpu.CompilerParams(
            dimension_semantics=("parallel",),
            vmem_limit_bytes=100 * 1024 * 1024),
    )(xcl, w1t, w2t, w3t, b1t, b2t, b3t)


def _mlp_kernel(x_ref, w1_ref, b1_ref, w2_ref, b2_ref, o_ref, acc):
    for i in range(7):
        d = jnp.dot(x_ref[:, i, :], w1_ref[i],
                    preferred_element_type=jnp.float32)
        if i == 0:
            acc[...] = d
        else:
            acc[...] = acc[...] + d
    h = jnp.maximum(acc[...] + b1_ref[...], 0.0).astype(jnp.bfloat16)
    o = jnp.dot(h, w2_ref[...], preferred_element_type=jnp.float32)
    o_ref[...] = o + b2_ref[...]


def _mlp(feat, w1m, b1, w2, b2):
    B = feat.shape[0]
    mt = min(MT, B)
    return pl.pallas_call(
        _mlp_kernel,
        out_shape=jax.ShapeDtypeStruct((B, 128), jnp.float32),
        grid=(B // mt,),
        in_specs=[
            pl.BlockSpec((mt, 7, 896), lambda b: (b, 0, 0)),
            pl.BlockSpec((7, 896, 128), lambda b: (0, 0, 0)),
            pl.BlockSpec((1, 128), lambda b: (0, 0)),
            pl.BlockSpec((128, 128), lambda b: (0, 0)),
            pl.BlockSpec((1, 128), lambda b: (0, 0)),
        ],
        out_specs=pl.BlockSpec((mt, 128), lambda b: (b, 0)),
        scratch_shapes=[pltpu.VMEM((mt, 128), jnp.float32)],
        compiler_params=pltpu.CompilerParams(
            dimension_semantics=("parallel",),
            vmem_limit_bytes=100 * 1024 * 1024),
    )(feat, w1m, b1, w2, b2)


def _toeplitz(wp, cin, cout, win, wout, kstride, cstride, interleave):
    eye = np.stack([np.eye(win, wout, k=1 - dx, dtype=np.float32)
                    for dx in range(3)])
    eye = jnp.asarray(eye)
    slabs = []
    for dy in range(3):
        w = wp[3 * dy:3 * dy + 3, :cin, :cout]
        t = jnp.einsum('duw,dcn->cunw', eye, w)
        if interleave:
            t = jnp.stack([t, jnp.zeros_like(t)], axis=2)
            t = t.reshape(cin, 2 * win, cout, wout)
        if cstride > t.shape[1]:
            t = jnp.pad(t, ((0, 0), (0, cstride - t.shape[1]), (0, 0), (0, 0)))
        slabs.append(t.reshape(cin * cstride, cout * wout))
    return jnp.stack(slabs).astype(jnp.bfloat16)


@jax.jit
def _forward(x_nchw, w1p, w2p, w3p, bstack, w1_fc, b1, w2_fc, b2):
    B = x_nchw.shape[0]
    w1t = _toeplitz(w1p, 3, 16, 56, 56, 1, 64, False)
    w2t = _toeplitz(w2p, 16, 32, 28, 28, 2, 56, True)
    w3t = _toeplitz(w3p, 32, 64, 14, 14, 2, 28, True)
    b1t = jnp.repeat(bstack[0, 0, 0, :16], 56).reshape(1, 896)
    b2t = jnp.repeat(bstack[1, 0, 0, :32], 28).reshape(1, 896)
    b3t = jnp.repeat(bstack[2, 0, 0, :64], 14).reshape(1, 896)
    f1 = w1_fc.reshape(7, 7, 128, 128)[:, :, :64, :]
    f1 = jnp.transpose(f1, (0, 2, 1, 3))
    f1 = jnp.stack([f1, jnp.zeros_like(f1)], axis=3)
    w1m = f1.reshape(7, 896, 128).astype(jnp.bfloat16)

    xcl = jnp.pad(x_nchw.astype(jnp.bfloat16),
                  ((0, 0), (0, 0), (1, 7), (0, 8)))
    xcl = xcl.reshape(B, 3, 8, 8, 64).transpose(0, 1, 3, 2, 4)
    feat = _conv_stack(xcl, w1t, w2t, w3t, b1t, b2t, b3t)
    out = _mlp(feat, w1m, b1, w2_fc.astype(jnp.bfloat16), b2)
    return out[:, :5]


def kernel(x_nchw, w1p, w2p, w3p, bstack, w1_fc, b1, w2_fc, b2):
    return _forward(x_nchw, w1p, w2p, w3p, bstack, w1_fc, b1, w2_fc, b2)


# speedup vs baseline: 1.4702x; 1.0181x over previous
import numpy as np
import jax
import jax.numpy as jnp
from jax.experimental import pallas as pl
from jax.experimental.pallas import tpu as pltpu

BT = 32
MT = 256


def _conv_kernel(x_ref, w1_ref, w2_ref, w3_ref, b1_ref, b2_ref, b3_ref,
                 o_ref, xt1, acc1, xt2, acc2, xt3, acc3):
    f32 = jnp.float32
    M = BT * 7

    for c in range(3):
        xt1[:, :, :, 64 * c:64 * (c + 1)] = x_ref[:, c, :, :, :]

    def conv(xt, nq, w_ref, acc, kdim):
        for dy in range(3):
            parts = []
            for q in range(nq):
                o = q + dy
                parts.append(xt[:, o % nq, o // nq:o // nq + 7, :])
            lhs = jnp.concatenate(parts, axis=0).reshape(nq * M, kdim)
            d = jnp.dot(lhs, w_ref[dy], preferred_element_type=f32)
            if dy == 0:
                acc[...] = d
            else:
                acc[...] = acc[...] + d
        return acc[...]

    def pool_bias_relu(a, c, bias):
        rp = jnp.maximum(a[2 * c * M:(2 * c + 1) * M, :],
                         a[(2 * c + 1) * M:(2 * c + 2) * M, :])
        sh = jnp.concatenate([rp[:, 1:], rp[:, :1]], axis=-1)
        wm = jnp.maximum(rp, sh)
        act = jnp.maximum(wm + bias, 0.0).astype(jnp.bfloat16)
        return act.reshape(BT, 7, 896)

    a1 = conv(xt1, 8, w1_ref, acc1, 192)
    xt2[:, 0, 0:1, :] = jnp.zeros((BT, 1, 896), jnp.bfloat16)
    xt2[:, 1, 7:8, :] = jnp.zeros((BT, 1, 896), jnp.bfloat16)
    for c in range(4):
        xt2[:, (c + 1) % 4, (c + 1) // 4:(c + 1) // 4 + 7, :] = \
            pool_bias_relu(a1, c, b1_ref[...])

    a2 = conv(xt2, 4, w2_ref, acc2, 896)
    xt3[:, 0, 0:1, :] = jnp.zeros((BT, 1, 896), jnp.bfloat16)
    xt3[:, 1, 7:8, :] = jnp.zeros((BT, 1, 896), jnp.bfloat16)
    for c in range(2):
        xt3[:, (c + 1) % 2, (c + 1) // 2:(c + 1) // 2 + 7, :] = \
            pool_bias_relu(a2, c, b2_ref[...])

    a3 = conv(xt3, 2, w3_ref, acc3, 896)
    o_ref[...] = pool_bias_relu(a3, 0, b3_ref[...])


def _conv_stack(xcl, w1t, w2t, w3t, b1t, b2t, b3t):
    B = xcl.shape[0]
    return pl.pallas_call(
        _conv_kernel,
        out_shape=jax.ShapeDtypeStruct((B, 7, 896), jnp.bfloat16),
        grid=(B // BT,),
        in_specs=[
            pl.BlockSpec((BT, 3, 8, 8, 64), lambda b: (b, 0, 0, 0, 0)),
            pl.BlockSpec((3, 192, 896), lambda b: (0, 0, 0)),
            pl.BlockSpec((3, 896, 896), lambda b: (0, 0, 0)),
            pl.BlockSpec((3, 896, 896), lambda b: (0, 0, 0)),
            pl.BlockSpec((1, 896), lambda b: (0, 0)),
            pl.BlockSpec((1, 896), lambda b: (0, 0)),
            pl.BlockSpec((1, 896), lambda b: (0, 0)),
        ],
        out_specs=pl.BlockSpec((BT, 7, 896), lambda b: (b, 0, 0)),
        scratch_shapes=[
            pltpu.VMEM((BT, 8, 8, 192), jnp.bfloat16),
            pltpu.VMEM((8 * BT * 7, 896), jnp.float32),
            pltpu.VMEM((BT, 4, 8, 896), jnp.bfloat16),
            pltpu.VMEM((4 * BT * 7, 896), jnp.float32),
            pltpu.VMEM((BT, 2, 8, 896), jnp.bfloat16),
            pltpu.VMEM((2 * BT * 7, 896), jnp.float32),
        ],
        compiler_params=pltpu.CompilerParams(
            dimension_semantics=("parallel",),
            vmem_limit_bytes=100 * 1024 * 1024),
    )(xcl, w1t, w2t, w3t, b1t, b2t, b3t)


def _mlp_kernel(x_ref, w1_ref, b1_ref, w2_ref, b2_ref, o_ref, acc):
    for i in range(7):
        d = jnp.dot(x_ref[:, i, :], w1_ref[i],
                    preferred_element_type=jnp.float32)
        if i == 0:
            acc[...] = d
        else:
            acc[...] = acc[...] + d
    h = jnp.maximum(acc[...] + b1_ref[...], 0.0).astype(jnp.bfloat16)
    o = jnp.dot(h, w2_ref[...], preferred_element_type=jnp.float32)
    o_ref[...] = o + b2_ref[...]


def _mlp(feat, w1m, b1, w2, b2):
    B = feat.shape[0]
    mt = min(MT, B)
    return pl.pallas_call(
        _mlp_kernel,
        out_shape=jax.ShapeDtypeStruct((B, 128), jnp.float32),
        grid=(B // mt,),
        in_specs=[
            pl.BlockSpec((mt, 7, 896), lambda b: (b, 0, 0)),
            pl.BlockSpec((7, 896, 128), lambda b: (0, 0, 0)),
            pl.BlockSpec((1, 128), lambda b: (0, 0)),
            pl.BlockSpec((128, 128), lambda b: (0, 0)),
            pl.BlockSpec((1, 128), lambda b: (0, 0)),
        ],
        out_specs=pl.BlockSpec((mt, 128), lambda b: (b, 0)),
        scratch_shapes=[pltpu.VMEM((mt, 128), jnp.float32)],
        compiler_params=pltpu.CompilerParams(
            dimension_semantics=("parallel",),
            vmem_limit_bytes=100 * 1024 * 1024),
    )(feat, w1m, b1, w2, b2)


def _toeplitz(wp, cin, cout, win, wout, kstride, cstride, interleave):
    eye = np.stack([np.eye(win, wout, k=1 - dx, dtype=np.float32)
                    for dx in range(3)])
    eye = jnp.asarray(eye)
    slabs = []
    for dy in range(3):
        w = wp[3 * dy:3 * dy + 3, :cin, :cout]
        t = jnp.einsum('duw,dcn->cunw', eye, w)
        if interleave:
            t = jnp.stack([t, jnp.zeros_like(t)], axis=2)
            t = t.reshape(cin, 2 * win, cout, wout)
        if cstride > t.shape[1]:
            t = jnp.pad(t, ((0, 0), (0, cstride - t.shape[1]), (0, 0), (0, 0)))
        slabs.append(t.reshape(cin * cstride, cout * wout))
    return jnp.stack(slabs).astype(jnp.bfloat16)


@jax.jit
def _forward(x_nchw, w1p, w2p, w3p, bstack, w1_fc, b1, w2_fc, b2):
    B = x_nchw.shape[0]
    w1t = _toeplitz(w1p, 3, 16, 56, 56, 1, 64, False)
    w2t = _toeplitz(w2p, 16, 32, 28, 28, 2, 56, True)
    w3t = _toeplitz(w3p, 32, 64, 14, 14, 2, 28, True)
    b1t = jnp.repeat(bstack[0, 0, 0, :16], 56).reshape(1, 896)
    b2t = jnp.repeat(bstack[1, 0, 0, :32], 28).reshape(1, 896)
    b3t = jnp.repeat(bstack[2, 0, 0, :64], 14).reshape(1, 896)
    f1 = w1_fc.reshape(7, 7, 128, 128)[:, :, :64, :]
    f1 = jnp.transpose(f1, (0, 2, 1, 3))
    f1 = jnp.stack([f1, jnp.zeros_like(f1)], axis=3)
    w1m = f1.reshape(7, 896, 128).astype(jnp.bfloat16)

    xcl = jnp.pad(x_nchw.astype(jnp.bfloat16),
                  ((0, 0), (0, 0), (1, 7), (0, 8)))
    xcl = xcl.reshape(B, 3, 8, 8, 64).transpose(0, 1, 3, 2, 4)
    feat = _conv_stack(xcl, w1t, w2t, w3t, b1t, b2t, b3t)
    out = _mlp(feat, w1m, b1, w2_fc.astype(jnp.bfloat16), b2)
    return out[:, :5]


def kernel(x_nchw, w1p, w2p, w3p, bstack, w1_fc, b1, w2_fc, b2):
    return _forward(x_nchw, w1p, w2p, w3p, bstack, w1_fc, b1, w2_fc, b2)
